# baseline (speedup 1.0000x reference)
import functools

import jax
import jax.numpy as jnp
from jax import lax
from jax.experimental import pallas as pl
from jax.experimental.pallas import tpu as pltpu
from jax.experimental.pallas import tpu_sc as plsc

N = 10000
E = 320000
DIN = 128
H1 = 100
H2 = 20
B = 64
DOUT = 10

NC = 2
NS = 16
NW = NC * NS

NPAD = 10240
RB = 1024
NBLK = NPAD // RB
D1 = 112
D2 = 32
ONES_COL = 100
DEG_COL = 20

CH = 64
EPT = E // NW
PPT = 240
EPT_PAD = EPT + PPT
NCHUNK = EPT_PAD // CH
RPT = NPAD // NS


def _matmul1_body(x_ref, w_ref, o_ref):
    y = jnp.dot(x_ref[...], w_ref[...], preferred_element_type=jnp.float32)
    col = lax.broadcasted_iota(jnp.int32, y.shape, 1)
    o_ref[...] = jnp.where(col == ONES_COL, 1.0, y)


def _mid_body(p0_ref, p1_ref, w2_ref, b1_ref, o_ref):
    a = p0_ref[...] + p1_ref[...]
    deg = a[:, ONES_COL:ONES_COL + 1]
    rdeg = 1.0 / jnp.maximum(deg, 1.0)
    h = jnp.maximum(a * rdeg + b1_ref[...], 0.0)
    y = jnp.dot(h, w2_ref[...], preferred_element_type=jnp.float32)
    col = lax.broadcasted_iota(jnp.int32, y.shape, 1)
    o_ref[...] = jnp.where(col == DEG_COL, deg, y)


def _sc_agg_body(tab_hbm, src_hbm, dst_hbm, zero_hbm, out_hbm,
                 src_v, dst_v, buf0, acc_sh, sem):
    c = lax.axis_index("c")
    s = lax.axis_index("s")
    w = s * NC + c
    r0 = s * RPT
    pltpu.sync_copy(zero_hbm.at[pl.ds(r0, RPT)], acc_sh.at[pl.ds(r0, RPT)])
    pltpu.sync_copy(src_hbm.at[w], src_v)
    pltpu.sync_copy(dst_hbm.at[w], dst_v)
    plsc.subcore_barrier()

    def body(j, carry):
        pltpu.async_copy(tab_hbm.at[src_v.at[j]], buf0, sem).wait()
        pltpu.sync_copy(buf0, acc_sh.at[dst_v.at[j]], add=True)
        return carry

    lax.fori_loop(0, NCHUNK, body, 0)
    plsc.subcore_barrier()
    pltpu.sync_copy(acc_sh.at[pl.ds(r0, RPT)], out_hbm.at[c, pl.ds(r0, RPT)])


def _sc_agg(table, srcp, dstp, zeros, d):
    mesh = plsc.VectorSubcoreMesh(core_axis_name="c", subcore_axis_name="s",
                                  num_cores=NC, num_subcores=NS)
    kern = pl.kernel(
        _sc_agg_body,
        out_type=jax.ShapeDtypeStruct((NC, NPAD, d), jnp.float32),
        mesh=mesh,
        scratch_types=[
            pltpu.VMEM((NCHUNK, CH), jnp.int32),
            pltpu.VMEM((NCHUNK, CH), jnp.int32),
            pltpu.VMEM((CH, d), jnp.float32),
            pltpu.VMEM_SHARED((NPAD, d), jnp.float32),
            pltpu.SemaphoreType.DMA,
        ],
        compiler_params=pltpu.CompilerParams(use_tc_tiling_on_sc=False),
    )
    return kern(table, srcp, dstp, zeros)


def _final_body(p0_ref, p1_ref, tab_ref, gid_ref, sf_ref, gw_ref, gb_ref,
                bw_ref, bb_ref, b2_ref, f1w_ref, f1b_ref, f2w_ref, f2b_ref,
                f3w_ref, f3b_ref, g1_ref, be1_ref, g2_ref, be2_ref,
                o_ref, hs_ref):
    i = pl.program_id(0)
    a = p0_ref[...] + p1_ref[...]
    deg = tab_ref[...][:, DEG_COL:DEG_COL + 1]
    rdeg = 1.0 / jnp.maximum(deg, 1.0)
    h2 = jnp.maximum(a * rdeg + b2_ref[...], 0.0)
    col = lax.broadcasted_iota(jnp.int32, h2.shape, 1)
    h2 = jnp.where(col == DEG_COL, 1.0, jnp.where(col > DEG_COL, 0.0, h2))
    gid = gid_ref[0, 0, :].astype(jnp.int32)
    onehot = (lax.broadcasted_iota(jnp.int32, (B, RB), 0)
              == gid[None, :]).astype(jnp.float32)
    part = jnp.dot(onehot, h2, preferred_element_type=jnp.float32)

    @pl.when(i == 0)
    def _():
        hs_ref[...] = part

    @pl.when(i > 0)
    def _():
        hs_ref[...] += part

    @pl.when(i == NBLK - 1)
    def _():
        hs = hs_ref[...]
        cnt = hs[:, DEG_COL:DEG_COL + 1]
        hg = hs[:, :H2] / jnp.maximum(cnt, 1.0)
        sf = sf_ref[...]
        glin = jnp.dot(sf, gw_ref[...], preferred_element_type=jnp.float32)
        gamma = 1.0 / (1.0 + jnp.exp(-(glin + gb_ref[...])))
        beta = jnp.dot(sf, bw_ref[...],
                       preferred_element_type=jnp.float32) + bb_ref[...]
        hg = hg * gamma + beta

        t = jnp.dot(hg, f1w_ref[...],
                    preferred_element_type=jnp.float32) + f1b_ref[...]
        m = jnp.mean(t, axis=0, keepdims=True)
        v = jnp.mean((t - m) * (t - m), axis=0, keepdims=True)
        t = (t - m) * lax.rsqrt(v + 1e-5) * g1_ref[...] + be1_ref[...]
        t = jnp.maximum(t, 0.0)

        t = jnp.dot(t, f2w_ref[...],
                    preferred_element_type=jnp.float32) + f2b_ref[...]
        m = jnp.mean(t, axis=0, keepdims=True)
        v = jnp.mean((t - m) * (t - m), axis=0, keepdims=True)
        t = (t - m) * lax.rsqrt(v + 1e-5) * g2_ref[...] + be2_ref[...]
        t = jnp.maximum(t, 0.0)

        o_ref[...] = jnp.dot(t, f3w_ref[...],
                             preferred_element_type=jnp.float32) + f3b_ref[...]


def kernel(x, edge_index, graph_ids, self_feat, W1, b1, W2, b2, gW, gb,
           bW, bb, fc1W, fc1b, fc2W, fc2b, fc3W, fc3b, bn1g, bn1b,
           bn2g, bn2b):
    f32 = jnp.float32
    src = edge_index[0].astype(jnp.int32).reshape(NW, EPT)
    dst = edge_index[1].astype(jnp.int32).reshape(NW, EPT)
    pad_src = jnp.full((NW, PPT), N, jnp.int32)
    pad_dst = jnp.broadcast_to(
        N + jnp.arange(NW, dtype=jnp.int32)[:, None], (NW, PPT))
    srcp = jnp.concatenate([src, pad_src], axis=1).reshape(NW, NCHUNK, CH)
    dstp = jnp.concatenate([dst, pad_dst], axis=1).reshape(NW, NCHUNK, CH)

    xp = jnp.zeros((NPAD, DIN), f32).at[:N].set(x)
    W1p = jnp.zeros((DIN, D1), f32).at[:, :H1].set(W1)
    b1p = jnp.zeros((1, D1), f32).at[0, :H1].set(b1)
    W2p = jnp.zeros((D1, D2), f32).at[:H1, :H2].set(W2)
    b2p = jnp.zeros((1, D2), f32).at[0, :H2].set(b2)
    zeros1 = jnp.zeros((NPAD, D1), f32)
    zeros2 = jnp.zeros((NPAD, D2), f32)
    gidf = jnp.concatenate(
        [graph_ids.astype(f32), jnp.full((NPAD - N,), float(B), f32)]
    ).reshape(NBLK, 1, RB)

    tab1 = pl.pallas_call(
        _matmul1_body,
        grid=(NBLK,),
        in_specs=[pl.BlockSpec((RB, DIN), lambda i: (i, 0)),
                  pl.BlockSpec((DIN, D1), lambda i: (0, 0))],
        out_specs=pl.BlockSpec((RB, D1), lambda i: (i, 0)),
        out_shape=jax.ShapeDtypeStruct((NPAD, D1), f32),
    )(xp, W1p)

    part1 = _sc_agg(tab1, srcp, dstp, zeros1, D1)

    tab2 = pl.pallas_call(
        _mid_body,
        grid=(NBLK,),
        in_specs=[pl.BlockSpec((RB, D1), lambda i: (i, 0)),
                  pl.BlockSpec((RB, D1), lambda i: (i, 0)),
                  pl.BlockSpec((D1, D2), lambda i: (0, 0)),
                  pl.BlockSpec((1, D1), lambda i: (0, 0))],
        out_specs=pl.BlockSpec((RB, D2), lambda i: (i, 0)),
        out_shape=jax.ShapeDtypeStruct((NPAD, D2), f32),
    )(part1[0], part1[1], W2p, b1p)

    part2 = _sc_agg(tab2, srcp, dstp, zeros2, D2)

    full = lambda shape: pl.BlockSpec(shape, lambda i: tuple(0 for _ in shape))
    out = pl.pallas_call(
        _final_body,
        grid=(NBLK,),
        in_specs=[pl.BlockSpec((RB, D2), lambda i: (i, 0)),
                  pl.BlockSpec((RB, D2), lambda i: (i, 0)),
                  pl.BlockSpec((RB, D2), lambda i: (i, 0)),
                  pl.BlockSpec((1, 1, RB), lambda i: (i, 0, 0)),
                  full((B, 16)), full((16, H2)), full((1, H2)),
                  full((16, H2)), full((1, H2)), full((1, D2)),
                  full((H2, 128)), full((1, 128)), full((128, 32)),
                  full((1, 32)), full((32, DOUT)), full((1, DOUT)),
                  full((1, 128)), full((1, 128)), full((1, 32)),
                  full((1, 32))],
        out_specs=pl.BlockSpec((B, DOUT), lambda i: (0, 0)),
        out_shape=jax.ShapeDtypeStruct((B, DOUT), f32),
        scratch_shapes=[pltpu.VMEM((B, D2), f32)],
    )(part2[0], part2[1], tab2, gidf, self_feat, gW, gb.reshape(1, -1),
      bW, bb.reshape(1, -1), b2p, fc1W, fc1b.reshape(1, -1), fc2W,
      fc2b.reshape(1, -1), fc3W, fc3b.reshape(1, -1), bn1g.reshape(1, -1),
      bn1b.reshape(1, -1), bn2g.reshape(1, -1), bn2b.reshape(1, -1))
    return out


# speedup vs baseline: 1.1383x; 1.1383x over previous
import functools

import jax
import jax.numpy as jnp
from jax import lax
from jax.experimental import pallas as pl
from jax.experimental.pallas import tpu as pltpu
from jax.experimental.pallas import tpu_sc as plsc

N = 10000
E = 320000
DIN = 128
H1 = 100
H2 = 20
B = 64
DOUT = 10

NC = 2
NS = 16
NW = NC * NS

NPAD = 10240
RB = 1024
NBLK = NPAD // RB
D1 = 112
D2 = 32
ONES_COL = 100
DEG_COL = 20

CH = 128
EPT = E // NW
PPT = 240
EPT_PAD = EPT + PPT
NCHUNK = EPT_PAD // CH
RPT = NPAD // NS


def _matmul1_body(x_ref, w_ref, o_ref):
    y = jnp.dot(x_ref[...], w_ref[...], preferred_element_type=jnp.float32)
    col = lax.broadcasted_iota(jnp.int32, y.shape, 1)
    o_ref[...] = jnp.where(col == ONES_COL, 1.0, y)


def _mid_body(p0_ref, p1_ref, w2_ref, b1_ref, o_ref):
    a = p0_ref[...] + p1_ref[...]
    deg = a[:, ONES_COL:ONES_COL + 1]
    rdeg = 1.0 / jnp.maximum(deg, 1.0)
    h = jnp.maximum(a * rdeg + b1_ref[...], 0.0)
    y = jnp.dot(h, w2_ref[...], preferred_element_type=jnp.float32)
    col = lax.broadcasted_iota(jnp.int32, y.shape, 1)
    o_ref[...] = jnp.where(col == DEG_COL, deg, y)


def _sc_agg_body(tab_hbm, src_hbm, dst_hbm, zero_hbm, out_hbm,
                 src_v, dst_v, buf0, acc_sh, sem):
    c = lax.axis_index("c")
    s = lax.axis_index("s")
    w = s * NC + c
    r0 = s * RPT
    pltpu.sync_copy(zero_hbm.at[pl.ds(r0, RPT)], acc_sh.at[pl.ds(r0, RPT)])
    pltpu.sync_copy(src_hbm.at[w], src_v)
    pltpu.sync_copy(dst_hbm.at[w], dst_v)
    plsc.subcore_barrier()

    def body(j, carry):
        pltpu.async_copy(tab_hbm.at[src_v.at[j]], buf0, sem).wait()
        pltpu.sync_copy(buf0, acc_sh.at[dst_v.at[j]], add=True)
        return carry

    lax.fori_loop(0, NCHUNK, body, 0)
    plsc.subcore_barrier()
    pltpu.sync_copy(acc_sh.at[pl.ds(r0, RPT)], out_hbm.at[c, pl.ds(r0, RPT)])


def _sc_agg(table, srcp, dstp, zeros, d):
    mesh = plsc.VectorSubcoreMesh(core_axis_name="c", subcore_axis_name="s",
                                  num_cores=NC, num_subcores=NS)
    kern = pl.kernel(
        _sc_agg_body,
        out_type=jax.ShapeDtypeStruct((NC, NPAD, d), jnp.float32),
        mesh=mesh,
        scratch_types=[
            pltpu.VMEM((NCHUNK, CH), jnp.int32),
            pltpu.VMEM((NCHUNK, CH), jnp.int32),
            pltpu.VMEM((CH, d), jnp.float32),
            pltpu.VMEM_SHARED((NPAD, d), jnp.float32),
            pltpu.SemaphoreType.DMA,
        ],
        compiler_params=pltpu.CompilerParams(use_tc_tiling_on_sc=False),
    )
    return kern(table, srcp, dstp, zeros)


def _final_body(p0_ref, p1_ref, tab_ref, gid_ref, sf_ref, gw_ref, gb_ref,
                bw_ref, bb_ref, b2_ref, f1w_ref, f1b_ref, f2w_ref, f2b_ref,
                f3w_ref, f3b_ref, g1_ref, be1_ref, g2_ref, be2_ref,
                o_ref, hs_ref):
    i = pl.program_id(0)
    a = p0_ref[...] + p1_ref[...]
    deg = tab_ref[...][:, DEG_COL:DEG_COL + 1]
    rdeg = 1.0 / jnp.maximum(deg, 1.0)
    h2 = jnp.maximum(a * rdeg + b2_ref[...], 0.0)
    col = lax.broadcasted_iota(jnp.int32, h2.shape, 1)
    h2 = jnp.where(col == DEG_COL, 1.0, jnp.where(col > DEG_COL, 0.0, h2))
    gid = gid_ref[0, 0, :].astype(jnp.int32)
    onehot = (lax.broadcasted_iota(jnp.int32, (B, RB), 0)
              == gid[None, :]).astype(jnp.float32)
    part = jnp.dot(onehot, h2, preferred_element_type=jnp.float32)

    @pl.when(i == 0)
    def _():
        hs_ref[...] = part

    @pl.when(i > 0)
    def _():
        hs_ref[...] += part

    @pl.when(i == NBLK - 1)
    def _():
        hs = hs_ref[...]
        cnt = hs[:, DEG_COL:DEG_COL + 1]
        hg = hs[:, :H2] / jnp.maximum(cnt, 1.0)
        sf = sf_ref[...]
        glin = jnp.dot(sf, gw_ref[...], preferred_element_type=jnp.float32)
        gamma = 1.0 / (1.0 + jnp.exp(-(glin + gb_ref[...])))
        beta = jnp.dot(sf, bw_ref[...],
                       preferred_element_type=jnp.float32) + bb_ref[...]
        hg = hg * gamma + beta

        t = jnp.dot(hg, f1w_ref[...],
                    preferred_element_type=jnp.float32) + f1b_ref[...]
        m = jnp.mean(t, axis=0, keepdims=True)
        v = jnp.mean((t - m) * (t - m), axis=0, keepdims=True)
        t = (t - m) * lax.rsqrt(v + 1e-5) * g1_ref[...] + be1_ref[...]
        t = jnp.maximum(t, 0.0)

        t = jnp.dot(t, f2w_ref[...],
                    preferred_element_type=jnp.float32) + f2b_ref[...]
        m = jnp.mean(t, axis=0, keepdims=True)
        v = jnp.mean((t - m) * (t - m), axis=0, keepdims=True)
        t = (t - m) * lax.rsqrt(v + 1e-5) * g2_ref[...] + be2_ref[...]
        t = jnp.maximum(t, 0.0)

        o_ref[...] = jnp.dot(t, f3w_ref[...],
                             preferred_element_type=jnp.float32) + f3b_ref[...]


def kernel(x, edge_index, graph_ids, self_feat, W1, b1, W2, b2, gW, gb,
           bW, bb, fc1W, fc1b, fc2W, fc2b, fc3W, fc3b, bn1g, bn1b,
           bn2g, bn2b):
    f32 = jnp.float32
    src = edge_index[0].astype(jnp.int32).reshape(NW, EPT)
    dst = edge_index[1].astype(jnp.int32).reshape(NW, EPT)
    pad_src = jnp.full((NW, PPT), N, jnp.int32)
    pad_dst = jnp.broadcast_to(
        N + jnp.arange(NW, dtype=jnp.int32)[:, None], (NW, PPT))
    srcp = jnp.concatenate([src, pad_src], axis=1).reshape(NW, NCHUNK, CH)
    dstp = jnp.concatenate([dst, pad_dst], axis=1).reshape(NW, NCHUNK, CH)

    xp = jnp.zeros((NPAD, DIN), f32).at[:N].set(x)
    W1p = jnp.zeros((DIN, D1), f32).at[:, :H1].set(W1)
    b1p = jnp.zeros((1, D1), f32).at[0, :H1].set(b1)
    W2p = jnp.zeros((D1, D2), f32).at[:H1, :H2].set(W2)
    b2p = jnp.zeros((1, D2), f32).at[0, :H2].set(b2)
    zeros1 = jnp.zeros((NPAD, D1), f32)
    zeros2 = jnp.zeros((NPAD, D2), f32)
    gidf = jnp.concatenate(
        [graph_ids.astype(f32), jnp.full((NPAD - N,), float(B), f32)]
    ).reshape(NBLK, 1, RB)

    tab1 = pl.pallas_call(
        _matmul1_body,
        grid=(NBLK,),
        in_specs=[pl.BlockSpec((RB, DIN), lambda i: (i, 0)),
                  pl.BlockSpec((DIN, D1), lambda i: (0, 0))],
        out_specs=pl.BlockSpec((RB, D1), lambda i: (i, 0)),
        out_shape=jax.ShapeDtypeStruct((NPAD, D1), f32),
    )(xp, W1p)

    part1 = _sc_agg(tab1, srcp, dstp, zeros1, D1)

    tab2 = pl.pallas_call(
        _mid_body,
        grid=(NBLK,),
        in_specs=[pl.BlockSpec((RB, D1), lambda i: (i, 0)),
                  pl.BlockSpec((RB, D1), lambda i: (i, 0)),
                  pl.BlockSpec((D1, D2), lambda i: (0, 0)),
                  pl.BlockSpec((1, D1), lambda i: (0, 0))],
        out_specs=pl.BlockSpec((RB, D2), lambda i: (i, 0)),
        out_shape=jax.ShapeDtypeStruct((NPAD, D2), f32),
    )(part1[0], part1[1], W2p, b1p)

    part2 = _sc_agg(tab2, srcp, dstp, zeros2, D2)

    full = lambda shape: pl.BlockSpec(shape, lambda i: tuple(0 for _ in shape))
    out = pl.pallas_call(
        _final_body,
        grid=(NBLK,),
        in_specs=[pl.BlockSpec((RB, D2), lambda i: (i, 0)),
                  pl.BlockSpec((RB, D2), lambda i: (i, 0)),
                  pl.BlockSpec((RB, D2), lambda i: (i, 0)),
                  pl.BlockSpec((1, 1, RB), lambda i: (i, 0, 0)),
                  full((B, 16)), full((16, H2)), full((1, H2)),
                  full((16, H2)), full((1, H2)), full((1, D2)),
                  full((H2, 128)), full((1, 128)), full((128, 32)),
                  full((1, 32)), full((32, DOUT)), full((1, DOUT)),
                  full((1, 128)), full((1, 128)), full((1, 32)),
                  full((1, 32))],
        out_specs=pl.BlockSpec((B, DOUT), lambda i: (0, 0)),
        out_shape=jax.ShapeDtypeStruct((B, DOUT), f32),
        scratch_shapes=[pltpu.VMEM((B, D2), f32)],
    )(part2[0], part2[1], tab2, gidf, self_feat, gW, gb.reshape(1, -1),
      bW, bb.reshape(1, -1), b2p, fc1W, fc1b.reshape(1, -1), fc2W,
      fc2b.reshape(1, -1), fc3W, fc3b.reshape(1, -1), bn1g.reshape(1, -1),
      bn1b.reshape(1, -1), bn2g.reshape(1, -1), bn2b.reshape(1, -1))
    return out


# speedup vs baseline: 1.5227x; 1.3376x over previous
import functools

import jax
import jax.numpy as jnp
from jax import lax
from jax.experimental import pallas as pl
from jax.experimental.pallas import tpu as pltpu
from jax.experimental.pallas import tpu_sc as plsc

N = 10000
E = 320000
DIN = 128
H1 = 100
H2 = 20
B = 64
DOUT = 10

NC = 2
NS = 16
NW = NC * NS

NPAD = 10240
RB = 1024
NBLK = NPAD // RB
D1 = 112
D2 = 32
ONES_COL = 100
DEG_COL = 20

CH = 128
EPT = E // NW
PPT = 112
EPT_PAD = EPT + PPT
NCHUNK = EPT_PAD // CH
RPT = NPAD // NS


def _matmul1_body(x_ref, w_ref, o_ref):
    y = jnp.dot(x_ref[...], w_ref[...], preferred_element_type=jnp.float32)
    col = lax.broadcasted_iota(jnp.int32, y.shape, 1)
    o_ref[...] = jnp.where(col == ONES_COL, 1.0, y)


def _mid_body(p0_ref, p1_ref, w2_ref, b1_ref, o_ref):
    a = p0_ref[...] + p1_ref[...]
    deg = a[:, ONES_COL:ONES_COL + 1]
    rdeg = 1.0 / jnp.maximum(deg, 1.0)
    h = jnp.maximum(a * rdeg + b1_ref[...], 0.0)
    y = jnp.dot(h, w2_ref[...], preferred_element_type=jnp.float32)
    col = lax.broadcasted_iota(jnp.int32, y.shape, 1)
    o_ref[...] = jnp.where(col == DEG_COL, deg, y)


def _sc_agg_body(tab_hbm, src_hbm, dst_hbm, zero_hbm, out_hbm,
                 src_v, dst_v, buf0, acc_sh, sem):
    c = lax.axis_index("c")
    s = lax.axis_index("s")
    w = s * NC + c
    r0 = s * RPT
    pltpu.sync_copy(zero_hbm.at[pl.ds(r0, RPT)], acc_sh.at[pl.ds(r0, RPT)])
    pltpu.sync_copy(src_hbm.at[w], src_v)
    pltpu.sync_copy(dst_hbm.at[w], dst_v)
    plsc.subcore_barrier()

    def body(j, carry):
        pltpu.async_copy(tab_hbm.at[src_v.at[j]], buf0, sem).wait()
        pltpu.sync_copy(buf0, acc_sh.at[dst_v.at[j]], add=True)
        return carry

    lax.fori_loop(0, NCHUNK, body, 0)
    plsc.subcore_barrier()
    pltpu.sync_copy(acc_sh.at[pl.ds(r0, RPT)], out_hbm.at[c, pl.ds(r0, RPT)])


def _sc_agg(table, srcp, dstp, zeros, d):
    mesh = plsc.VectorSubcoreMesh(core_axis_name="c", subcore_axis_name="s",
                                  num_cores=NC, num_subcores=NS)
    kern = pl.kernel(
        _sc_agg_body,
        out_type=jax.ShapeDtypeStruct((NC, NPAD, d), jnp.float32),
        mesh=mesh,
        scratch_types=[
            pltpu.VMEM((NCHUNK, CH), jnp.int32),
            pltpu.VMEM((NCHUNK, CH), jnp.int32),
            pltpu.VMEM((CH, d), jnp.float32),
            pltpu.VMEM_SHARED((NPAD, d), jnp.float32),
            pltpu.SemaphoreType.DMA,
        ],
        compiler_params=pltpu.CompilerParams(use_tc_tiling_on_sc=False),
    )
    return kern(table, srcp, dstp, zeros)


def _final_body(p0_ref, p1_ref, tab_ref, gid_ref, sf_ref, gw_ref, gb_ref,
                bw_ref, bb_ref, b2_ref, f1w_ref, f1b_ref, f2w_ref, f2b_ref,
                f3w_ref, f3b_ref, g1_ref, be1_ref, g2_ref, be2_ref,
                o_ref, hs_ref):
    i = pl.program_id(0)
    a = p0_ref[...] + p1_ref[...]
    deg = tab_ref[...][:, DEG_COL:DEG_COL + 1]
    rdeg = 1.0 / jnp.maximum(deg, 1.0)
    h2 = jnp.maximum(a * rdeg + b2_ref[...], 0.0)
    col = lax.broadcasted_iota(jnp.int32, h2.shape, 1)
    h2 = jnp.where(col == DEG_COL, 1.0, jnp.where(col > DEG_COL, 0.0, h2))
    gid = gid_ref[0, 0, :].astype(jnp.int32)
    onehot = (lax.broadcasted_iota(jnp.int32, (B, RB), 0)
              == gid[None, :]).astype(jnp.float32)
    part = jnp.dot(onehot, h2, preferred_element_type=jnp.float32)

    @pl.when(i == 0)
    def _():
        hs_ref[...] = part

    @pl.when(i > 0)
    def _():
        hs_ref[...] += part

    @pl.when(i == NBLK - 1)
    def _():
        hs = hs_ref[...]
        cnt = hs[:, DEG_COL:DEG_COL + 1]
        hg = hs[:, :H2] / jnp.maximum(cnt, 1.0)
        sf = sf_ref[...]
        glin = jnp.dot(sf, gw_ref[...], preferred_element_type=jnp.float32)
        gamma = 1.0 / (1.0 + jnp.exp(-(glin + gb_ref[...])))
        beta = jnp.dot(sf, bw_ref[...],
                       preferred_element_type=jnp.float32) + bb_ref[...]
        hg = hg * gamma + beta

        t = jnp.dot(hg, f1w_ref[...],
                    preferred_element_type=jnp.float32) + f1b_ref[...]
        m = jnp.mean(t, axis=0, keepdims=True)
        v = jnp.mean((t - m) * (t - m), axis=0, keepdims=True)
        t = (t - m) * lax.rsqrt(v + 1e-5) * g1_ref[...] + be1_ref[...]
        t = jnp.maximum(t, 0.0)

        t = jnp.dot(t, f2w_ref[...],
                    preferred_element_type=jnp.float32) + f2b_ref[...]
        m = jnp.mean(t, axis=0, keepdims=True)
        v = jnp.mean((t - m) * (t - m), axis=0, keepdims=True)
        t = (t - m) * lax.rsqrt(v + 1e-5) * g2_ref[...] + be2_ref[...]
        t = jnp.maximum(t, 0.0)

        o_ref[...] = jnp.dot(t, f3w_ref[...],
                             preferred_element_type=jnp.float32) + f3b_ref[...]


def kernel(x, edge_index, graph_ids, self_feat, W1, b1, W2, b2, gW, gb,
           bW, bb, fc1W, fc1b, fc2W, fc2b, fc3W, fc3b, bn1g, bn1b,
           bn2g, bn2b):
    f32 = jnp.float32
    src = edge_index[0].astype(jnp.int32).reshape(NW, EPT)
    dst = edge_index[1].astype(jnp.int32).reshape(NW, EPT)
    pad_src = jnp.full((NW, PPT), N, jnp.int32)
    pad_dst = jnp.broadcast_to(
        N + jnp.arange(NW, dtype=jnp.int32)[:, None], (NW, PPT))
    srcp = jnp.concatenate([src, pad_src], axis=1).reshape(NW, NCHUNK, CH)
    dstp = jnp.concatenate([dst, pad_dst], axis=1).reshape(NW, NCHUNK, CH)

    xp = jnp.zeros((NPAD, DIN), f32).at[:N].set(x)
    W1p = jnp.zeros((DIN, D1), f32).at[:, :H1].set(W1)
    b1p = jnp.zeros((1, D1), f32).at[0, :H1].set(b1)
    W2p = jnp.zeros((D1, D2), f32).at[:H1, :H2].set(W2)
    b2p = jnp.zeros((1, D2), f32).at[0, :H2].set(b2)
    zeros1 = jnp.zeros((NPAD, D1), f32)
    zeros2 = jnp.zeros((NPAD, D2), f32)
    gidf = jnp.concatenate(
        [graph_ids.astype(f32), jnp.full((NPAD - N,), float(B), f32)]
    ).reshape(NBLK, 1, RB)

    tab1 = pl.pallas_call(
        _matmul1_body,
        grid=(NBLK,),
        in_specs=[pl.BlockSpec((RB, DIN), lambda i: (i, 0)),
                  pl.BlockSpec((DIN, D1), lambda i: (0, 0))],
        out_specs=pl.BlockSpec((RB, D1), lambda i: (i, 0)),
        out_shape=jax.ShapeDtypeStruct((NPAD, D1), f32),
    )(xp, W1p)

    part1 = _sc_agg(tab1, srcp, dstp, zeros1, D1)

    tab2 = pl.pallas_call(
        _mid_body,
        grid=(NBLK,),
        in_specs=[pl.BlockSpec((RB, D1), lambda i: (i, 0)),
                  pl.BlockSpec((RB, D1), lambda i: (i, 0)),
                  pl.BlockSpec((D1, D2), lambda i: (0, 0)),
                  pl.BlockSpec((1, D1), lambda i: (0, 0))],
        out_specs=pl.BlockSpec((RB, D2), lambda i: (i, 0)),
        out_shape=jax.ShapeDtypeStruct((NPAD, D2), f32),
    )(part1[0], part1[1], W2p, b1p)

    part2 = _sc_agg(tab2, srcp, dstp, zeros2, D2)

    full = lambda shape: pl.BlockSpec(shape, lambda i: tuple(0 for _ in shape))
    out = pl.pallas_call(
        _final_body,
        grid=(NBLK,),
        in_specs=[pl.BlockSpec((RB, D2), lambda i: (i, 0)),
                  pl.BlockSpec((RB, D2), lambda i: (i, 0)),
                  pl.BlockSpec((RB, D2), lambda i: (i, 0)),
                  pl.BlockSpec((1, 1, RB), lambda i: (i, 0, 0)),
                  full((B, 16)), full((16, H2)), full((1, H2)),
                  full((16, H2)), full((1, H2)), full((1, D2)),
                  full((H2, 128)), full((1, 128)), full((128, 32)),
                  full((1, 32)), full((32, DOUT)), full((1, DOUT)),
                  full((1, 128)), full((1, 128)), full((1, 32)),
                  full((1, 32))],
        out_specs=pl.BlockSpec((B, DOUT), lambda i: (0, 0)),
        out_shape=jax.ShapeDtypeStruct((B, DOUT), f32),
        scratch_shapes=[pltpu.VMEM((B, D2), f32)],
    )(part2[0], part2[1], tab2, gidf, self_feat, gW, gb.reshape(1, -1),
      bW, bb.reshape(1, -1), b2p, fc1W, fc1b.reshape(1, -1), fc2W,
      fc2b.reshape(1, -1), fc3W, fc3b.reshape(1, -1), bn1g.reshape(1, -1),
      bn1b.reshape(1, -1), bn2g.reshape(1, -1), bn2b.reshape(1, -1))
    return out


# speedup vs baseline: 1.7150x; 1.1263x over previous
import functools

import jax
import jax.numpy as jnp
from jax import lax
from jax.experimental import pallas as pl
from jax.experimental.pallas import tpu as pltpu
from jax.experimental.pallas import tpu_sc as plsc

N = 10000
E = 320000
DIN = 128
H1 = 100
H2 = 20
B = 64
DOUT = 10

NC = 2
NS = 16
NW = NC * NS

NPAD = 10240
RB = 1024
NBLK = NPAD // RB
D1 = 112
D2 = 32
ONES_COL = 100
DEG_COL = 20

CH = 128
EPT = E // NW
PPT = 112
EPT_PAD = EPT + PPT
NCHUNK = EPT_PAD // CH
RPT = NPAD // NS


def _matmul1_body(x_ref, w_ref, o_ref):
    y = jnp.dot(x_ref[...], w_ref[...], preferred_element_type=jnp.float32)
    col = lax.broadcasted_iota(jnp.int32, y.shape, 1)
    o_ref[...] = jnp.where(col == ONES_COL, 1.0, y)


def _mid_body(p0_ref, p1_ref, w2_ref, b1_ref, o_ref):
    a = p0_ref[...] + p1_ref[...]
    deg = a[:, ONES_COL:ONES_COL + 1]
    rdeg = 1.0 / jnp.maximum(deg, 1.0)
    h = jnp.maximum(a * rdeg + b1_ref[...], 0.0)
    y = jnp.dot(h, w2_ref[...], preferred_element_type=jnp.float32)
    col = lax.broadcasted_iota(jnp.int32, y.shape, 1)
    o_ref[...] = jnp.where(col == DEG_COL, deg, y)


def _sc_agg_body(tab_hbm, src_hbm, dst_hbm, zero_hbm, out_hbm,
                 src_v, dst_v, buf0, buf1, acc_sh, sem):
    c = lax.axis_index("c")
    s = lax.axis_index("s")
    w = s * NC + c
    r0 = s * RPT
    pltpu.sync_copy(zero_hbm.at[pl.ds(r0, RPT)], acc_sh.at[pl.ds(r0, RPT)])
    pltpu.sync_copy(src_hbm.at[w], src_v)
    pltpu.sync_copy(dst_hbm.at[w], dst_v)
    plsc.subcore_barrier()

    pltpu.async_copy(tab_hbm.at[src_v.at[0]], buf0, sem)

    def body(k, carry):
        j0 = 2 * k
        j1 = j0 + 1
        pltpu.make_async_copy(tab_hbm.at[src_v.at[j0]], buf0, sem).wait()
        pltpu.async_copy(tab_hbm.at[src_v.at[j1]], buf1, sem)
        pltpu.sync_copy(buf0, acc_sh.at[dst_v.at[j0]], add=True)
        pltpu.make_async_copy(tab_hbm.at[src_v.at[j1]], buf1, sem).wait()
        pltpu.async_copy(tab_hbm.at[src_v.at[j0 + 2]], buf0, sem)
        pltpu.sync_copy(buf1, acc_sh.at[dst_v.at[j1]], add=True)
        return carry

    lax.fori_loop(0, NCHUNK // 2, body, 0)
    pltpu.make_async_copy(tab_hbm.at[src_v.at[NCHUNK - 1]], buf0, sem).wait()
    pltpu.sync_copy(buf0, acc_sh.at[dst_v.at[NCHUNK - 1]], add=True)
    plsc.subcore_barrier()
    pltpu.sync_copy(acc_sh.at[pl.ds(r0, RPT)], out_hbm.at[c, pl.ds(r0, RPT)])


def _sc_agg(table, srcp, dstp, zeros, d):
    mesh = plsc.VectorSubcoreMesh(core_axis_name="c", subcore_axis_name="s",
                                  num_cores=NC, num_subcores=NS)
    kern = pl.kernel(
        _sc_agg_body,
        out_type=jax.ShapeDtypeStruct((NC, NPAD, d), jnp.float32),
        mesh=mesh,
        scratch_types=[
            pltpu.VMEM((NCHUNK, CH), jnp.int32),
            pltpu.VMEM((NCHUNK, CH), jnp.int32),
            pltpu.VMEM((CH, d), jnp.float32),
            pltpu.VMEM((CH, d), jnp.float32),
            pltpu.VMEM_SHARED((NPAD, d), jnp.float32),
            pltpu.SemaphoreType.DMA,
        ],
        compiler_params=pltpu.CompilerParams(use_tc_tiling_on_sc=False),
    )
    return kern(table, srcp, dstp, zeros)


def _final_body(p0_ref, p1_ref, tab_ref, gid_ref, sf_ref, gw_ref, gb_ref,
                bw_ref, bb_ref, b2_ref, f1w_ref, f1b_ref, f2w_ref, f2b_ref,
                f3w_ref, f3b_ref, g1_ref, be1_ref, g2_ref, be2_ref,
                o_ref, hs_ref):
    i = pl.program_id(0)
    a = p0_ref[...] + p1_ref[...]
    deg = tab_ref[...][:, DEG_COL:DEG_COL + 1]
    rdeg = 1.0 / jnp.maximum(deg, 1.0)
    h2 = jnp.maximum(a * rdeg + b2_ref[...], 0.0)
    col = lax.broadcasted_iota(jnp.int32, h2.shape, 1)
    h2 = jnp.where(col == DEG_COL, 1.0, jnp.where(col > DEG_COL, 0.0, h2))
    gid = gid_ref[0, 0, :].astype(jnp.int32)
    onehot = (lax.broadcasted_iota(jnp.int32, (B, RB), 0)
              == gid[None, :]).astype(jnp.float32)
    part = jnp.dot(onehot, h2, preferred_element_type=jnp.float32)

    @pl.when(i == 0)
    def _():
        hs_ref[...] = part

    @pl.when(i > 0)
    def _():
        hs_ref[...] += part

    @pl.when(i == NBLK - 1)
    def _():
        hs = hs_ref[...]
        cnt = hs[:, DEG_COL:DEG_COL + 1]
        hg = hs[:, :H2] / jnp.maximum(cnt, 1.0)
        sf = sf_ref[...]
        glin = jnp.dot(sf, gw_ref[...], preferred_element_type=jnp.float32)
        gamma = 1.0 / (1.0 + jnp.exp(-(glin + gb_ref[...])))
        beta = jnp.dot(sf, bw_ref[...],
                       preferred_element_type=jnp.float32) + bb_ref[...]
        hg = hg * gamma + beta

        t = jnp.dot(hg, f1w_ref[...],
                    preferred_element_type=jnp.float32) + f1b_ref[...]
        m = jnp.mean(t, axis=0, keepdims=True)
        v = jnp.mean((t - m) * (t - m), axis=0, keepdims=True)
        t = (t - m) * lax.rsqrt(v + 1e-5) * g1_ref[...] + be1_ref[...]
        t = jnp.maximum(t, 0.0)

        t = jnp.dot(t, f2w_ref[...],
                    preferred_element_type=jnp.float32) + f2b_ref[...]
        m = jnp.mean(t, axis=0, keepdims=True)
        v = jnp.mean((t - m) * (t - m), axis=0, keepdims=True)
        t = (t - m) * lax.rsqrt(v + 1e-5) * g2_ref[...] + be2_ref[...]
        t = jnp.maximum(t, 0.0)

        o_ref[...] = jnp.dot(t, f3w_ref[...],
                             preferred_element_type=jnp.float32) + f3b_ref[...]


def kernel(x, edge_index, graph_ids, self_feat, W1, b1, W2, b2, gW, gb,
           bW, bb, fc1W, fc1b, fc2W, fc2b, fc3W, fc3b, bn1g, bn1b,
           bn2g, bn2b):
    f32 = jnp.float32
    src = edge_index[0].astype(jnp.int32).reshape(NW, EPT)
    dst = edge_index[1].astype(jnp.int32).reshape(NW, EPT)
    pad_src = jnp.full((NW, PPT), N, jnp.int32)
    pad_dst = jnp.broadcast_to(
        N + jnp.arange(NW, dtype=jnp.int32)[:, None], (NW, PPT))
    srcp = jnp.concatenate([src, pad_src], axis=1).reshape(NW, NCHUNK, CH)
    dstp = jnp.concatenate([dst, pad_dst], axis=1).reshape(NW, NCHUNK, CH)

    xp = jnp.zeros((NPAD, DIN), f32).at[:N].set(x)
    W1p = jnp.zeros((DIN, D1), f32).at[:, :H1].set(W1)
    b1p = jnp.zeros((1, D1), f32).at[0, :H1].set(b1)
    W2p = jnp.zeros((D1, D2), f32).at[:H1, :H2].set(W2)
    b2p = jnp.zeros((1, D2), f32).at[0, :H2].set(b2)
    zeros1 = jnp.zeros((NPAD, D1), f32)
    zeros2 = jnp.zeros((NPAD, D2), f32)
    gidf = jnp.concatenate(
        [graph_ids.astype(f32), jnp.full((NPAD - N,), float(B), f32)]
    ).reshape(NBLK, 1, RB)

    tab1 = pl.pallas_call(
        _matmul1_body,
        grid=(NBLK,),
        in_specs=[pl.BlockSpec((RB, DIN), lambda i: (i, 0)),
                  pl.BlockSpec((DIN, D1), lambda i: (0, 0))],
        out_specs=pl.BlockSpec((RB, D1), lambda i: (i, 0)),
        out_shape=jax.ShapeDtypeStruct((NPAD, D1), f32),
    )(xp, W1p)

    part1 = _sc_agg(tab1, srcp, dstp, zeros1, D1)

    tab2 = pl.pallas_call(
        _mid_body,
        grid=(NBLK,),
        in_specs=[pl.BlockSpec((RB, D1), lambda i: (i, 0)),
                  pl.BlockSpec((RB, D1), lambda i: (i, 0)),
                  pl.BlockSpec((D1, D2), lambda i: (0, 0)),
                  pl.BlockSpec((1, D1), lambda i: (0, 0))],
        out_specs=pl.BlockSpec((RB, D2), lambda i: (i, 0)),
        out_shape=jax.ShapeDtypeStruct((NPAD, D2), f32),
    )(part1[0], part1[1], W2p, b1p)

    part2 = _sc_agg(tab2, srcp, dstp, zeros2, D2)

    full = lambda shape: pl.BlockSpec(shape, lambda i: tuple(0 for _ in shape))
    out = pl.pallas_call(
        _final_body,
        grid=(NBLK,),
        in_specs=[pl.BlockSpec((RB, D2), lambda i: (i, 0)),
                  pl.BlockSpec((RB, D2), lambda i: (i, 0)),
                  pl.BlockSpec((RB, D2), lambda i: (i, 0)),
                  pl.BlockSpec((1, 1, RB), lambda i: (i, 0, 0)),
                  full((B, 16)), full((16, H2)), full((1, H2)),
                  full((16, H2)), full((1, H2)), full((1, D2)),
                  full((H2, 128)), full((1, 128)), full((128, 32)),
                  full((1, 32)), full((32, DOUT)), full((1, DOUT)),
                  full((1, 128)), full((1, 128)), full((1, 32)),
                  full((1, 32))],
        out_specs=pl.BlockSpec((B, DOUT), lambda i: (0, 0)),
        out_shape=jax.ShapeDtypeStruct((B, DOUT), f32),
        scratch_shapes=[pltpu.VMEM((B, D2), f32)],
    )(part2[0], part2[1], tab2, gidf, self_feat, gW, gb.reshape(1, -1),
      bW, bb.reshape(1, -1), b2p, fc1W, fc1b.reshape(1, -1), fc2W,
      fc2b.reshape(1, -1), fc3W, fc3b.reshape(1, -1), bn1g.reshape(1, -1),
      bn1b.reshape(1, -1), bn2g.reshape(1, -1), bn2b.reshape(1, -1))
    return out


# speedup vs baseline: 1.8416x; 1.0738x over previous
import functools

import jax
import jax.numpy as jnp
from jax import lax
from jax.experimental import pallas as pl
from jax.experimental.pallas import tpu as pltpu
from jax.experimental.pallas import tpu_sc as plsc

N = 10000
E = 320000
DIN = 128
H1 = 100
H2 = 20
B = 64
DOUT = 10

NC = 2
NS = 16
NW = NC * NS

NPAD = 10240
RB = 1024
NBLK = NPAD // RB
D1 = 112
D2 = 32
ONES_COL = 100
DEG_COL = 20

CH = 128
CH2 = 632
EPT = E // NW
PPT = 112
EPT_PAD = EPT + PPT
NCHUNK = EPT_PAD // CH
RPT = NPAD // NS


def _matmul1_body(x_ref, w_ref, o_ref):
    y = jnp.dot(x_ref[...], w_ref[...], preferred_element_type=jnp.float32)
    col = lax.broadcasted_iota(jnp.int32, y.shape, 1)
    o_ref[...] = jnp.where(col == ONES_COL, 1.0, y)


def _mid_body(p0_ref, p1_ref, w2_ref, b1_ref, o_ref):
    a = p0_ref[...] + p1_ref[...]
    deg = a[:, ONES_COL:ONES_COL + 1]
    rdeg = 1.0 / jnp.maximum(deg, 1.0)
    h = jnp.maximum(a * rdeg + b1_ref[...], 0.0)
    y = jnp.dot(h, w2_ref[...], preferred_element_type=jnp.float32)
    col = lax.broadcasted_iota(jnp.int32, y.shape, 1)
    o_ref[...] = jnp.where(col == DEG_COL, deg, y)


def _sc_agg_body(nchunk, tab_hbm, src_hbm, dst_hbm, zero_hbm, out_hbm,
                 src_v, dst_v, buf0, buf1, acc_sh, sem):
    c = lax.axis_index("c")
    s = lax.axis_index("s")
    w = s * NC + c
    r0 = s * RPT
    pltpu.sync_copy(zero_hbm.at[pl.ds(r0, RPT)], acc_sh.at[pl.ds(r0, RPT)])
    pltpu.sync_copy(src_hbm.at[w], src_v)
    pltpu.sync_copy(dst_hbm.at[w], dst_v)
    plsc.subcore_barrier()

    pltpu.async_copy(tab_hbm.at[src_v.at[0]], buf0, sem)

    def body(k, carry):
        j0 = 2 * k
        j1 = j0 + 1
        pltpu.make_async_copy(tab_hbm.at[src_v.at[j0]], buf0, sem).wait()
        pltpu.async_copy(tab_hbm.at[src_v.at[j1]], buf1, sem)
        pltpu.sync_copy(buf0, acc_sh.at[dst_v.at[j0]], add=True)
        pltpu.make_async_copy(tab_hbm.at[src_v.at[j1]], buf1, sem).wait()

        @pl.when(j0 + 2 < nchunk)
        def _():
            pltpu.async_copy(tab_hbm.at[src_v.at[j0 + 2]], buf0, sem)

        pltpu.sync_copy(buf1, acc_sh.at[dst_v.at[j1]], add=True)
        return carry

    lax.fori_loop(0, nchunk // 2, body, 0)
    if nchunk % 2:
        pltpu.make_async_copy(tab_hbm.at[src_v.at[nchunk - 1]], buf0, sem).wait()
        pltpu.sync_copy(buf0, acc_sh.at[dst_v.at[nchunk - 1]], add=True)
    plsc.subcore_barrier()
    pltpu.sync_copy(acc_sh.at[pl.ds(r0, RPT)], out_hbm.at[c, pl.ds(r0, RPT)])


def _sc_agg(table, srcp, dstp, zeros, d, ch):
    nchunk = EPT_PAD // ch
    mesh = plsc.VectorSubcoreMesh(core_axis_name="c", subcore_axis_name="s",
                                  num_cores=NC, num_subcores=NS)
    kern = pl.kernel(
        functools.partial(_sc_agg_body, nchunk),
        out_type=jax.ShapeDtypeStruct((NC, NPAD, d), jnp.float32),
        mesh=mesh,
        scratch_types=[
            pltpu.VMEM((nchunk, ch), jnp.int32),
            pltpu.VMEM((nchunk, ch), jnp.int32),
            pltpu.VMEM((ch, d), jnp.float32),
            pltpu.VMEM((ch, d), jnp.float32),
            pltpu.VMEM_SHARED((NPAD, d), jnp.float32),
            pltpu.SemaphoreType.DMA,
        ],
        compiler_params=pltpu.CompilerParams(use_tc_tiling_on_sc=False),
    )
    return kern(table, srcp.reshape(NW, nchunk, ch),
                dstp.reshape(NW, nchunk, ch), zeros)


def _final_body(p0_ref, p1_ref, tab_ref, gid_ref, sf_ref, gw_ref, gb_ref,
                bw_ref, bb_ref, b2_ref, f1w_ref, f1b_ref, f2w_ref, f2b_ref,
                f3w_ref, f3b_ref, g1_ref, be1_ref, g2_ref, be2_ref,
                o_ref, hs_ref):
    i = pl.program_id(0)
    a = p0_ref[...] + p1_ref[...]
    deg = tab_ref[...][:, DEG_COL:DEG_COL + 1]
    rdeg = 1.0 / jnp.maximum(deg, 1.0)
    h2 = jnp.maximum(a * rdeg + b2_ref[...], 0.0)
    col = lax.broadcasted_iota(jnp.int32, h2.shape, 1)
    h2 = jnp.where(col == DEG_COL, 1.0, jnp.where(col > DEG_COL, 0.0, h2))
    gid = gid_ref[0, 0, :].astype(jnp.int32)
    onehot = (lax.broadcasted_iota(jnp.int32, (B, RB), 0)
              == gid[None, :]).astype(jnp.float32)
    part = jnp.dot(onehot, h2, preferred_element_type=jnp.float32)

    @pl.when(i == 0)
    def _():
        hs_ref[...] = part

    @pl.when(i > 0)
    def _():
        hs_ref[...] += part

    @pl.when(i == NBLK - 1)
    def _():
        hs = hs_ref[...]
        cnt = hs[:, DEG_COL:DEG_COL + 1]
        hg = hs[:, :H2] / jnp.maximum(cnt, 1.0)
        sf = sf_ref[...]
        glin = jnp.dot(sf, gw_ref[...], preferred_element_type=jnp.float32)
        gamma = 1.0 / (1.0 + jnp.exp(-(glin + gb_ref[...])))
        beta = jnp.dot(sf, bw_ref[...],
                       preferred_element_type=jnp.float32) + bb_ref[...]
        hg = hg * gamma + beta

        t = jnp.dot(hg, f1w_ref[...],
                    preferred_element_type=jnp.float32) + f1b_ref[...]
        m = jnp.mean(t, axis=0, keepdims=True)
        v = jnp.mean((t - m) * (t - m), axis=0, keepdims=True)
        t = (t - m) * lax.rsqrt(v + 1e-5) * g1_ref[...] + be1_ref[...]
        t = jnp.maximum(t, 0.0)

        t = jnp.dot(t, f2w_ref[...],
                    preferred_element_type=jnp.float32) + f2b_ref[...]
        m = jnp.mean(t, axis=0, keepdims=True)
        v = jnp.mean((t - m) * (t - m), axis=0, keepdims=True)
        t = (t - m) * lax.rsqrt(v + 1e-5) * g2_ref[...] + be2_ref[...]
        t = jnp.maximum(t, 0.0)

        o_ref[...] = jnp.dot(t, f3w_ref[...],
                             preferred_element_type=jnp.float32) + f3b_ref[...]


def kernel(x, edge_index, graph_ids, self_feat, W1, b1, W2, b2, gW, gb,
           bW, bb, fc1W, fc1b, fc2W, fc2b, fc3W, fc3b, bn1g, bn1b,
           bn2g, bn2b):
    f32 = jnp.float32
    src = edge_index[0].astype(jnp.int32).reshape(NW, EPT)
    dst = edge_index[1].astype(jnp.int32).reshape(NW, EPT)
    pad_src = jnp.full((NW, PPT), N, jnp.int32)
    pad_dst = jnp.broadcast_to(
        N + jnp.arange(NW, dtype=jnp.int32)[:, None], (NW, PPT))
    srcp = jnp.concatenate([src, pad_src], axis=1)
    dstp = jnp.concatenate([dst, pad_dst], axis=1)

    xp = jnp.zeros((NPAD, DIN), f32).at[:N].set(x)
    W1p = jnp.zeros((DIN, D1), f32).at[:, :H1].set(W1)
    b1p = jnp.zeros((1, D1), f32).at[0, :H1].set(b1)
    W2p = jnp.zeros((D1, D2), f32).at[:H1, :H2].set(W2)
    b2p = jnp.zeros((1, D2), f32).at[0, :H2].set(b2)
    zeros1 = jnp.zeros((NPAD, D1), f32)
    zeros2 = jnp.zeros((NPAD, D2), f32)
    gidf = jnp.concatenate(
        [graph_ids.astype(f32), jnp.full((NPAD - N,), float(B), f32)]
    ).reshape(NBLK, 1, RB)

    tab1 = pl.pallas_call(
        _matmul1_body,
        grid=(NBLK,),
        in_specs=[pl.BlockSpec((RB, DIN), lambda i: (i, 0)),
                  pl.BlockSpec((DIN, D1), lambda i: (0, 0))],
        out_specs=pl.BlockSpec((RB, D1), lambda i: (i, 0)),
        out_shape=jax.ShapeDtypeStruct((NPAD, D1), f32),
    )(xp, W1p)

    part1 = _sc_agg(tab1, srcp, dstp, zeros1, D1, CH)

    tab2 = pl.pallas_call(
        _mid_body,
        grid=(NBLK,),
        in_specs=[pl.BlockSpec((RB, D1), lambda i: (i, 0)),
                  pl.BlockSpec((RB, D1), lambda i: (i, 0)),
                  pl.BlockSpec((D1, D2), lambda i: (0, 0)),
                  pl.BlockSpec((1, D1), lambda i: (0, 0))],
        out_specs=pl.BlockSpec((RB, D2), lambda i: (i, 0)),
        out_shape=jax.ShapeDtypeStruct((NPAD, D2), f32),
    )(part1[0], part1[1], W2p, b1p)

    part2 = _sc_agg(tab2, srcp, dstp, zeros2, D2, CH2)

    full = lambda shape: pl.BlockSpec(shape, lambda i: tuple(0 for _ in shape))
    out = pl.pallas_call(
        _final_body,
        grid=(NBLK,),
        in_specs=[pl.BlockSpec((RB, D2), lambda i: (i, 0)),
                  pl.BlockSpec((RB, D2), lambda i: (i, 0)),
                  pl.BlockSpec((RB, D2), lambda i: (i, 0)),
                  pl.BlockSpec((1, 1, RB), lambda i: (i, 0, 0)),
                  full((B, 16)), full((16, H2)), full((1, H2)),
                  full((16, H2)), full((1, H2)), full((1, D2)),
                  full((H2, 128)), full((1, 128)), full((128, 32)),
                  full((1, 32)), full((32, DOUT)), full((1, DOUT)),
                  full((1, 128)), full((1, 128)), full((1, 32)),
                  full((1, 32))],
        out_specs=pl.BlockSpec((B, DOUT), lambda i: (0, 0)),
        out_shape=jax.ShapeDtypeStruct((B, DOUT), f32),
        scratch_shapes=[pltpu.VMEM((B, D2), f32)],
    )(part2[0], part2[1], tab2, gidf, self_feat, gW, gb.reshape(1, -1),
      bW, bb.reshape(1, -1), b2p, fc1W, fc1b.reshape(1, -1), fc2W,
      fc2b.reshape(1, -1), fc3W, fc3b.reshape(1, -1), bn1g.reshape(1, -1),
      bn1b.reshape(1, -1), bn2g.reshape(1, -1), bn2b.reshape(1, -1))
    return out
